# Initial kernel scaffold; baseline (speedup 1.0000x reference)
import jax
import jax.numpy as jnp
from jax.experimental import pallas as pl


def kernel(x, wg, rg, ru, rd, sg, su, sd):
    raise NotImplementedError("write your pallas kernel here")


# speedup vs baseline: 1.1827x; 1.1827x over previous
import functools

import jax
import jax.numpy as jnp
from jax import lax
from jax.experimental import pallas as pl
from jax.experimental.pallas import tpu as pltpu

_TT = 256
_RM = 256
_SM = 1024
_SF = 256

_pcall = functools.partial(pl.pallas_call)


def _silu(v):
    return v / (1.0 + jnp.exp(-v))


def _router_body(x_ref, wg_ref, gates_ref, idx_ref, imp_ref):
    m = pl.program_id(0)
    e_dim = wg_ref.shape[0]
    mt = x_ref.shape[0]
    x = x_ref[...]
    wg = wg_ref[...]
    clean = lax.dot_general(x, wg, (((1,), (1,)), ((), ())),
                            preferred_element_type=jnp.float32)
    neg = jnp.float32(-1e30)
    best1 = jnp.full((mt, 1), neg, jnp.float32)
    idx1 = jnp.zeros((mt, 1), jnp.int32)
    for e in range(e_dim):
        v = clean[:, e:e + 1]
        better = v > best1
        best1 = jnp.where(better, v, best1)
        idx1 = jnp.where(better, e, idx1)
    best2 = jnp.full((mt, 1), neg, jnp.float32)
    idx2 = jnp.zeros((mt, 1), jnp.int32)
    for e in range(e_dim):
        v = jnp.where(idx1 == e, neg, clean[:, e:e + 1])
        better = v > best2
        best2 = jnp.where(better, v, best2)
        idx2 = jnp.where(better, e, idx2)
    g2 = 1.0 / (1.0 + jnp.exp(best1 - best2))
    g1 = 1.0 - g2
    gates_ref[...] = jnp.concatenate(
        [g1, g2, jnp.zeros((mt, 126), jnp.float32)], axis=1)
    idx_ref[...] = jnp.concatenate(
        [idx1, idx2, jnp.zeros((mt, 126), jnp.int32)], axis=1)
    mx = jnp.max(clean, axis=1, keepdims=True)
    p = jnp.exp(clean - mx)
    p = p / jnp.sum(p, axis=1, keepdims=True)
    part = jnp.sum(p, axis=0, keepdims=True)
    imp_ref[pl.ds(m, 1), :] = jnp.concatenate(
        [part, jnp.zeros((1, 128 - e_dim), jnp.float32)], axis=1)


def _routed_body(em_ref, vm_ref, xs_ref, rg_ref, ru_ref, rd_ref, wb_ref,
                 ys_ref):
    t = pl.program_id(0)

    @pl.when(vm_ref[t] == 1)
    def _():
        xv = xs_ref[...]
        rgv = rg_ref[0]
        ruv = ru_ref[0]
        rdv = rd_ref[0]
        hg = lax.dot_general(xv, rgv, (((1,), (1,)), ((), ())),
                             preferred_element_type=jnp.float32)
        hu = lax.dot_general(xv, ruv, (((1,), (1,)), ((), ())),
                             preferred_element_type=jnp.float32)
        h = (_silu(hg) * hu).astype(jnp.bfloat16)
        yv = lax.dot_general(h, rdv, (((1,), (1,)), ((), ())),
                             preferred_element_type=jnp.float32)
        ys_ref[...] = yv * wb_ref[:, 0:1]


def _shared_body(xb_ref, sg_ref, su_ref, sd_ref, o_ref):
    f = pl.program_id(1)
    xv = xb_ref[...]
    sgv = sg_ref[...].astype(jnp.bfloat16)
    suv = su_ref[...].astype(jnp.bfloat16)
    sdv = sd_ref[...].astype(jnp.bfloat16)
    hg = lax.dot_general(xv, sgv, (((1,), (1,)), ((), ())),
                         preferred_element_type=jnp.float32)
    hu = lax.dot_general(xv, suv, (((1,), (1,)), ((), ())),
                         preferred_element_type=jnp.float32)
    h = (_silu(hg) * hu).astype(jnp.bfloat16)
    yv = lax.dot_general(h, sdv, (((1,), (1,)), ((), ())),
                         preferred_element_type=jnp.float32)

    @pl.when(f == 0)
    def _():
        o_ref[...] = yv

    @pl.when(f != 0)
    def _():
        o_ref[...] += yv


def kernel(x, wg, rg, ru, rd, sg, su, sd):
    b, t_dim, d = x.shape
    n = b * t_dim
    e_dim = wg.shape[0]
    dri = rg.shape[1]
    dsi = sg.shape[0]
    k_top = 2

    xf = x.reshape(n, d)
    xb = xf.astype(jnp.bfloat16)

    nt = n * k_top // _TT + e_dim
    pmax = nt * _TT

    gm = n // _RM
    gates_o, idx_o, imp_o = _pcall(
        _router_body,
        out_shape=[
            jax.ShapeDtypeStruct((n, 128), jnp.float32),
            jax.ShapeDtypeStruct((n, 128), jnp.int32),
            jax.ShapeDtypeStruct((gm, 128), jnp.float32),
        ],
        grid=(gm,),
        in_specs=[
            pl.BlockSpec((_RM, d), lambda m: (m, 0)),
            pl.BlockSpec((e_dim, d), lambda m: (0, 0)),
        ],
        out_specs=[
            pl.BlockSpec((_RM, 128), lambda m: (m, 0)),
            pl.BlockSpec((_RM, 128), lambda m: (m, 0)),
            pl.BlockSpec((gm, 128), lambda m: (0, 0)),
        ],
    )(xf, wg)
    imp = jnp.sum(imp_o[:, :e_dim], axis=0)
    ce = imp / n * e_dim
    lb_loss = jnp.mean(ce * ce)

    flat_e = jnp.stack([idx_o[:, 0], idx_o[:, 1]], axis=1).reshape(-1)
    flat_w = jnp.stack([gates_o[:, 0], gates_o[:, 1]], axis=1).reshape(-1)
    oh = (flat_e[:, None] == jnp.arange(e_dim, dtype=jnp.int32)
          ).astype(jnp.int32)
    csum = jnp.cumsum(oh, axis=0)
    counts = csum[-1]
    rank = jnp.take_along_axis(csum, flat_e[:, None], axis=1)[:, 0] - 1
    padded = ((counts + _TT - 1) // _TT) * _TT
    cum_p = jnp.cumsum(padded)
    poff = cum_p - padded
    dest = poff[flat_e] + rank
    tok = jnp.arange(n * k_top, dtype=jnp.int32) // k_top
    tok_buf = jnp.zeros((pmax,), jnp.int32).at[dest].set(tok)
    w_buf = jnp.zeros((pmax,), jnp.float32).at[dest].set(flat_w)
    pos = dest.reshape(n, k_top)
    tile_starts = jnp.arange(nt, dtype=jnp.int32) * _TT
    emap = jnp.minimum(
        jnp.searchsorted(cum_p, tile_starts, side='right').astype(jnp.int32),
        e_dim - 1)
    vmask = (tile_starts < cum_p[-1]).astype(jnp.int32)

    xs = xb[tok_buf]

    rgb = rg.astype(jnp.bfloat16)
    rub = ru.astype(jnp.bfloat16)
    rdb = rd.astype(jnp.bfloat16)
    wb = jnp.broadcast_to(w_buf[:, None], (pmax, 128))
    ys = _pcall(
        _routed_body,
        out_shape=jax.ShapeDtypeStruct((pmax, d), jnp.float32),
        grid_spec=pltpu.PrefetchScalarGridSpec(
            num_scalar_prefetch=2,
            grid=(nt,),
            in_specs=[
                pl.BlockSpec((_TT, d), lambda t, em, vm: (t, 0)),
                pl.BlockSpec((1, dri, d), lambda t, em, vm: (em[t], 0, 0)),
                pl.BlockSpec((1, dri, d), lambda t, em, vm: (em[t], 0, 0)),
                pl.BlockSpec((1, d, dri), lambda t, em, vm: (em[t], 0, 0)),
                pl.BlockSpec((_TT, 128), lambda t, em, vm: (t, 0)),
            ],
            out_specs=pl.BlockSpec((_TT, d), lambda t, em, vm: (t, 0)),
        ),
    )(emap, vmask, xs, rgb, rub, rdb, wb)

    gms = n // _SM
    gfs = dsi // _SF
    shared_out = _pcall(
        _shared_body,
        out_shape=jax.ShapeDtypeStruct((n, d), jnp.float32),
        grid=(gms, gfs),
        in_specs=[
            pl.BlockSpec((_SM, d), lambda m, f: (m, 0)),
            pl.BlockSpec((_SF, d), lambda m, f: (f, 0)),
            pl.BlockSpec((_SF, d), lambda m, f: (f, 0)),
            pl.BlockSpec((d, _SF), lambda m, f: (0, f)),
        ],
        out_specs=pl.BlockSpec((_SM, d), lambda m, f: (m, 0)),
        compiler_params=pltpu.CompilerParams(
            dimension_semantics=("parallel", "arbitrary")),
    )(xb, sg, su, sd)

    out = shared_out + ys[pos[:, 0]] + ys[pos[:, 1]]
    return out.reshape(b, t_dim, d), lb_loss


# speedup vs baseline: 1.3404x; 1.1334x over previous
import functools

import jax
import jax.numpy as jnp
from jax import lax
from jax.experimental import pallas as pl
from jax.experimental.pallas import tpu as pltpu

_TT = 256
_RM = 256
_SM = 1024
_SF = 512

_pcall = functools.partial(pl.pallas_call)


def _silu(v):
    return v / (1.0 + jnp.exp(-v))


def _router_body(x_ref, wg_ref, gates_ref, idx_ref, imp_ref):
    m = pl.program_id(0)
    e_dim = wg_ref.shape[0]
    mt = x_ref.shape[0]
    x = x_ref[...]
    wg = wg_ref[...]
    clean = lax.dot_general(x, wg, (((1,), (1,)), ((), ())),
                            preferred_element_type=jnp.float32)
    neg = jnp.float32(-1e30)
    best1 = jnp.full((mt, 1), neg, jnp.float32)
    idx1 = jnp.zeros((mt, 1), jnp.int32)
    for e in range(e_dim):
        v = clean[:, e:e + 1]
        better = v > best1
        best1 = jnp.where(better, v, best1)
        idx1 = jnp.where(better, e, idx1)
    best2 = jnp.full((mt, 1), neg, jnp.float32)
    idx2 = jnp.zeros((mt, 1), jnp.int32)
    for e in range(e_dim):
        v = jnp.where(idx1 == e, neg, clean[:, e:e + 1])
        better = v > best2
        best2 = jnp.where(better, v, best2)
        idx2 = jnp.where(better, e, idx2)
    g2 = 1.0 / (1.0 + jnp.exp(best1 - best2))
    g1 = 1.0 - g2
    gates_ref[...] = jnp.concatenate(
        [g1, g2, jnp.zeros((mt, 126), jnp.float32)], axis=1)
    idx_ref[...] = jnp.concatenate(
        [idx1, idx2, jnp.zeros((mt, 126), jnp.int32)], axis=1)
    mx = jnp.max(clean, axis=1, keepdims=True)
    p = jnp.exp(clean - mx)
    p = p / jnp.sum(p, axis=1, keepdims=True)
    part = jnp.sum(p, axis=0, keepdims=True)
    imp_ref[pl.ds(m, 1), :] = jnp.concatenate(
        [part, jnp.zeros((1, 128 - e_dim), jnp.float32)], axis=1)


def _routed_body(em_ref, vm_ref, xs_ref, rg_ref, ru_ref, rd_ref, wb_ref,
                 ys_ref):
    t = pl.program_id(0)

    @pl.when(vm_ref[t] == 1)
    def _():
        xv = xs_ref[...]
        rgv = rg_ref[0].astype(jnp.bfloat16)
        ruv = ru_ref[0].astype(jnp.bfloat16)
        rdv = rd_ref[0].astype(jnp.bfloat16)
        hg = lax.dot_general(xv, rgv, (((1,), (1,)), ((), ())),
                             preferred_element_type=jnp.float32)
        hu = lax.dot_general(xv, ruv, (((1,), (1,)), ((), ())),
                             preferred_element_type=jnp.float32)
        h = (_silu(hg) * hu).astype(jnp.bfloat16)
        yv = lax.dot_general(h, rdv, (((1,), (1,)), ((), ())),
                             preferred_element_type=jnp.float32)
        ys_ref[...] = yv * wb_ref[:, 0:1]


def _shared_body(xb_ref, sg_ref, su_ref, sd_ref, o_ref):
    f = pl.program_id(1)
    xv = xb_ref[...]
    sgv = sg_ref[...].astype(jnp.bfloat16)
    suv = su_ref[...].astype(jnp.bfloat16)
    sdv = sd_ref[...].astype(jnp.bfloat16)
    hg = lax.dot_general(xv, sgv, (((1,), (1,)), ((), ())),
                         preferred_element_type=jnp.float32)
    hu = lax.dot_general(xv, suv, (((1,), (1,)), ((), ())),
                         preferred_element_type=jnp.float32)
    h = (_silu(hg) * hu).astype(jnp.bfloat16)
    yv = lax.dot_general(h, sdv, (((1,), (1,)), ((), ())),
                         preferred_element_type=jnp.float32)

    @pl.when(f == 0)
    def _():
        o_ref[...] = yv

    @pl.when(f != 0)
    def _():
        o_ref[...] += yv


def kernel(x, wg, rg, ru, rd, sg, su, sd):
    b, t_dim, d = x.shape
    n = b * t_dim
    e_dim = wg.shape[0]
    dri = rg.shape[1]
    dsi = sg.shape[0]
    k_top = 2

    xf = x.reshape(n, d)
    xb = xf.astype(jnp.bfloat16)

    nt = n * k_top // _TT + e_dim
    pmax = nt * _TT

    gm = n // _RM
    gates_o, idx_o, imp_o = _pcall(
        _router_body,
        out_shape=[
            jax.ShapeDtypeStruct((n, 128), jnp.float32),
            jax.ShapeDtypeStruct((n, 128), jnp.int32),
            jax.ShapeDtypeStruct((gm, 128), jnp.float32),
        ],
        grid=(gm,),
        in_specs=[
            pl.BlockSpec((_RM, d), lambda m: (m, 0)),
            pl.BlockSpec((e_dim, d), lambda m: (0, 0)),
        ],
        out_specs=[
            pl.BlockSpec((_RM, 128), lambda m: (m, 0)),
            pl.BlockSpec((_RM, 128), lambda m: (m, 0)),
            pl.BlockSpec((gm, 128), lambda m: (0, 0)),
        ],
    )(xf, wg)
    imp = jnp.sum(imp_o[:, :e_dim], axis=0)
    ce = imp / n * e_dim
    lb_loss = jnp.mean(ce * ce)

    flat_e = jnp.stack([idx_o[:, 0], idx_o[:, 1]], axis=1).reshape(-1)
    flat_w = jnp.stack([gates_o[:, 0], gates_o[:, 1]], axis=1).reshape(-1)
    oh = (flat_e[:, None] == jnp.arange(e_dim, dtype=jnp.int32)
          ).astype(jnp.int32)
    csum = jnp.cumsum(oh, axis=0)
    counts = csum[-1]
    rank = jnp.take_along_axis(csum, flat_e[:, None], axis=1)[:, 0] - 1
    padded = ((counts + _TT - 1) // _TT) * _TT
    cum_p = jnp.cumsum(padded)
    poff = cum_p - padded
    dest = poff[flat_e] + rank
    tok = jnp.arange(n * k_top, dtype=jnp.int32) // k_top
    tok_buf = jnp.zeros((pmax,), jnp.int32).at[dest].set(tok)
    w_buf = jnp.zeros((pmax,), jnp.float32).at[dest].set(flat_w)
    pos = dest.reshape(n, k_top)
    tile_starts = jnp.arange(nt, dtype=jnp.int32) * _TT
    emap = jnp.minimum(
        jnp.searchsorted(cum_p, tile_starts, side='right').astype(jnp.int32),
        e_dim - 1)
    vmask = (tile_starts < cum_p[-1]).astype(jnp.int32)

    xs = xb[tok_buf]

    wb = jnp.broadcast_to(w_buf[:, None], (pmax, 128))
    ys = _pcall(
        _routed_body,
        out_shape=jax.ShapeDtypeStruct((pmax, d), jnp.float32),
        grid_spec=pltpu.PrefetchScalarGridSpec(
            num_scalar_prefetch=2,
            grid=(nt,),
            in_specs=[
                pl.BlockSpec((_TT, d), lambda t, em, vm: (t, 0)),
                pl.BlockSpec((1, dri, d), lambda t, em, vm: (em[t], 0, 0)),
                pl.BlockSpec((1, dri, d), lambda t, em, vm: (em[t], 0, 0)),
                pl.BlockSpec((1, d, dri), lambda t, em, vm: (em[t], 0, 0)),
                pl.BlockSpec((_TT, 128), lambda t, em, vm: (t, 0)),
            ],
            out_specs=pl.BlockSpec((_TT, d), lambda t, em, vm: (t, 0)),
        ),
    )(emap, vmask, xs, rg, ru, rd, wb)

    gms = n // _SM
    gfs = dsi // _SF
    shared_out = _pcall(
        _shared_body,
        out_shape=jax.ShapeDtypeStruct((n, d), jnp.float32),
        grid=(gms, gfs),
        in_specs=[
            pl.BlockSpec((_SM, d), lambda m, f: (m, 0)),
            pl.BlockSpec((_SF, d), lambda m, f: (f, 0)),
            pl.BlockSpec((_SF, d), lambda m, f: (f, 0)),
            pl.BlockSpec((d, _SF), lambda m, f: (0, f)),
        ],
        out_specs=pl.BlockSpec((_SM, d), lambda m, f: (m, 0)),
        compiler_params=pltpu.CompilerParams(
            dimension_semantics=("parallel", "arbitrary")),
    )(xb, sg, su, sd)

    out = shared_out + ys[pos[:, 0]] + ys[pos[:, 1]]
    return out.reshape(b, t_dim, d), lb_loss
